# baseline (speedup 1.0000x reference)
import dataclasses

import jax
import jax.numpy as jnp
from jax import lax
from jax.experimental import pallas as pl
from jax.experimental.pallas import tpu as pltpu
from jax.experimental.pallas import tpu_sc as plsc

N = 10000
E = 320000
D = 128
LAM = 0.5

NC = 2
NS = 16
NW = NC * NS
CHUNK = 128
E_PAD = 327680
EDGES_PER_TILE = E_PAD // NW
CHUNKS_PER_TILE = EDGES_PER_TILE // CHUNK
NPAD = N + 8
ZSTEP = 624
ZSPAN = 640

ROW_BLOCK = 1000


def _mlp_body(h_ref, w_ref, b_ref, z_ref):
    z = lax.dot_general(
        h_ref[...], w_ref[...], (((1,), (0,)), ((), ())),
        preferred_element_type=jnp.float32,
        precision=lax.Precision.HIGHEST,
    )
    z = jnp.maximum(z + b_ref[...], 0.0)
    nrm = jnp.sqrt(jnp.sum(z * z, axis=1, keepdims=True))
    z_ref[...] = z / jnp.maximum(nrm, 1e-12)


def _mlp(h, W, b2d):
    return pl.pallas_call(
        _mlp_body,
        grid=(N // ROW_BLOCK,),
        in_specs=[
            pl.BlockSpec((ROW_BLOCK, D), lambda i: (i, 0)),
            pl.BlockSpec((D, D), lambda i: (0, 0)),
            pl.BlockSpec((1, D), lambda i: (0, 0)),
        ],
        out_specs=pl.BlockSpec((ROW_BLOCK, D), lambda i: (i, 0)),
        out_shape=jax.ShapeDtypeStruct((N, D), jnp.float32),
    )(h, W, b2d)


CHG = 1
GW = CHG * CHUNK
NG = CHUNKS_PER_TILE // CHG


def _make_hop(with_deg):
    mesh = plsc.VectorSubcoreMesh(core_axis_name="c", subcore_axis_name="s")

    out_type = [jax.ShapeDtypeStruct((NC, N, D), jnp.float32)]
    scratch = [
        pltpu.VMEM((1, CHUNK), jnp.int32),
        pltpu.VMEM((1, CHUNK), jnp.int32),
        pltpu.VMEM((1, CHUNK), jnp.int32),
        pltpu.VMEM((1, CHUNK), jnp.int32),
        pltpu.VMEM((1, CHUNK), jnp.int32),
        pltpu.VMEM((1, CHUNK), jnp.int32),
        pltpu.VMEM((1, CHUNK), jnp.int32),
        pltpu.VMEM((1, CHUNK), jnp.int32),
        pltpu.VMEM((GW, D), jnp.float32),
        pltpu.VMEM((GW, D), jnp.float32),
        pltpu.VMEM_SHARED((NPAD, D), jnp.float32),
    ]
    if with_deg:
        out_type.append(jax.ShapeDtypeStruct((NW, 8, NPAD), jnp.float32))
        scratch.append(pltpu.VMEM((NPAD,), jnp.float32))
    scratch += [pltpu.SemaphoreType.DMA] * 7

    def inner(table, src3, dst3, zrows, out, degout, refs):
        (i0, i1, i2, i3, d0, d1, d2, d3, r0, r1, acc, degt,
         si0, si1, si2, si3, sg, ss0, ss1) = refs
        idxs = [i0, i1, i2, i3]
        idxd = [d0, d1, d2, d3]
        rows = [r0, r1]
        sem_i = [si0, si1, si2, si3]
        sem_s = [ss0, ss1]

        c = lax.axis_index("c")
        s = lax.axis_index("s")
        w = c * NS + s
        row0 = pl.multiple_of(s * ZSTEP, 8)
        gbase = w * NG
        pltpu.sync_copy(zrows, acc.at[pl.ds(row0, ZSPAN)])
        if with_deg:
            @pl.loop(0, NPAD // 16)
            def _(i):
                degt[pl.ds(pl.multiple_of(i * 16, 16), 16)] = jnp.zeros(
                    (16,), jnp.float32)
        for k in (0, 1):
            pltpu.async_copy(src3.at[gbase + k], idxs[k], sem_i[k])
            pltpu.async_copy(dst3.at[gbase + k], idxd[k], sem_i[k])
        plsc.subcore_barrier()

        @pl.loop(0, NG, step=4)
        def _(g):
            for k in range(4):
                gk = g + k
                rb = k % 2
                sl = k % 4

                @pl.when(gk >= 2)
                def _():
                    pltpu.make_async_copy(table.at[pl.ds(0, GW)],
                                          rows[rb], sem_s[rb]).wait()

                pltpu.make_async_copy(src3.at[0], idxs[sl],
                                      sem_i[sl]).wait()
                pltpu.make_async_copy(src3.at[0], idxd[sl],
                                      sem_i[sl]).wait()

                h = pltpu.async_copy(table.at[idxs[sl].at[0]], rows[rb], sg)

                @pl.when(gk + 2 < NG)
                def _():
                    nxt = gbase + gk + 2
                    pltpu.async_copy(src3.at[nxt], idxs[(sl + 2) % 4],
                                     sem_i[(sl + 2) % 4])
                    pltpu.async_copy(dst3.at[nxt], idxd[(sl + 2) % 4],
                                     sem_i[(sl + 2) % 4])

                if with_deg:
                    for t in range(CHUNK // 16):
                        iv = idxd[sl][0, pl.ds(t * 16, 16)]
                        plsc.addupdate_scatter(degt, [iv],
                                               jnp.ones((16,), jnp.float32))

                h.wait()
                pltpu.async_copy(rows[rb], acc.at[idxd[sl].at[0]],
                                 sem_s[rb], add=True)

        pltpu.make_async_copy(table.at[pl.ds(0, GW)], rows[0], sem_s[0]).wait()
        pltpu.make_async_copy(table.at[pl.ds(0, GW)], rows[1], sem_s[1]).wait()
        plsc.subcore_barrier()
        pltpu.sync_copy(acc.at[pl.ds(row0, ZSPAN)],
                        out.at[c, pl.ds(row0, ZSPAN)])
        if with_deg:
            pltpu.sync_copy(degt, degout.at[w, 0])

    if with_deg:
        def body(table, src3, dst3, zrows, out, degout, *refs):
            inner(table, src3, dst3, zrows, out, degout, refs)
    else:
        def body(table, src3, dst3, zrows, out, *refs):
            refs = refs[:11] + (None,) + refs[11:]
            inner(table, src3, dst3, zrows, out, None, refs)

    cp = pltpu.CompilerParams()
    if "needs_layout_passes" in pltpu.CompilerParams.__dataclass_fields__:
        cp = dataclasses.replace(cp, needs_layout_passes=False)
    return pl.kernel(body, out_type=out_type, mesh=mesh,
                     scratch_types=scratch, compiler_params=cp)


_hop_deg = _make_hop(True)
_hop = _make_hop(False)


def _c1_body(p_ref, pd_ref, out_ref):
    s = p_ref[0] + p_ref[1]
    deg = jnp.sum(pd_ref[:, 0, :], axis=0)[:N]
    out_ref[...] = s / jnp.maximum(deg, 1.0)[:, None]


def _combine1(p, pdeg):
    return pl.pallas_call(
        _c1_body,
        grid=(1,),
        in_specs=[
            pl.BlockSpec((NC, N, D), lambda i: (0, 0, 0)),
            pl.BlockSpec((NW, 8, NPAD), lambda i: (0, 0, 0)),
        ],
        out_specs=pl.BlockSpec((N, D), lambda i: (0, 0)),
        out_shape=jax.ShapeDtypeStruct((N, D), jnp.float32),
    )(p, pdeg)


def _c2_body(n1_ref, p_ref, pd_ref, out_ref):
    s = p_ref[0] + p_ref[1]
    deg = jnp.sum(pd_ref[:, 0, :], axis=0)[:N]
    neigh2 = s / jnp.maximum(deg, 1.0)[:, None]
    out_ref[...] = LAM * n1_ref[...] + (1.0 - LAM) * neigh2


def _combine2(n1, p, pdeg):
    return pl.pallas_call(
        _c2_body,
        grid=(1,),
        in_specs=[
            pl.BlockSpec((N, D), lambda i: (0, 0)),
            pl.BlockSpec((NC, N, D), lambda i: (0, 0, 0)),
            pl.BlockSpec((NW, 8, NPAD), lambda i: (0, 0, 0)),
        ],
        out_specs=pl.BlockSpec((N, D), lambda i: (0, 0)),
        out_shape=jax.ShapeDtypeStruct((N, D), jnp.float32),
    )(n1, p, pdeg)


def kernel(h, edge_index, W, b):
    z = _mlp(h, W, b.reshape(1, D))
    pad = E_PAD - E
    srcp = jnp.concatenate([edge_index[0],
                            jnp.zeros((pad,), jnp.int32)])
    dstp = jnp.concatenate([edge_index[1],
                            jnp.full((pad,), N, jnp.int32)])
    src3 = srcp.reshape(E_PAD // GW, CHG, CHUNK)
    dst3 = dstp.reshape(E_PAD // GW, CHG, CHUNK)
    zrows = jnp.zeros((ZSPAN, D), jnp.float32)
    p1, pdeg = _hop_deg(z, src3, dst3, zrows)
    neigh1 = _combine1(p1, pdeg)
    (p2,) = _hop(neigh1, src3, dst3, zrows)
    result = _combine2(neigh1, p2, pdeg)
    return (z, result)
